# baseline (speedup 1.0000x reference)
import functools
import math

import jax
import jax.numpy as jnp
import numpy as np
from jax import lax
from jax.experimental import pallas as pl
from jax.experimental.pallas import tpu as pltpu
from jax.experimental.pallas import tpu_sc as plsc

EPS = 1e-15
N_RBF = 20
CUTOFF = 5.0
FEAT = 128
N_NODES = 10000
N_EDGES = 160000

NB_PAD = 24
EDGE_BLK = 1280
GW = 40
SW = 40
N_SUBCORES = 16
N_CORES = 2
N_WORKERS = N_CORES * N_SUBCORES
TBL = 4 * FEAT
N_PAD = 10240
CH0 = 80640
CH1 = 79360

_vector_mesh = plsc.VectorSubcoreMesh(
    core_axis_name="core", subcore_axis_name="subcore")


def _start(src, dst, sem, add=False):
    pltpu.make_async_copy(src, dst, sem).start(add=add)


def _wait(src, dst, sem):
    pltpu.make_async_copy(src, dst, sem).wait()


TBLP = TBL // 2


def _make_gather(n_edges):
    nwin = n_edges // (N_WORKERS * GW)
    @functools.partial(
        pl.kernel,
        out_type=jax.ShapeDtypeStruct((n_edges, TBLP), jnp.int32),
        mesh=_vector_mesh,
        scratch_types=(
            [pltpu.VMEM((nwin, GW), jnp.int32)]
            + [pltpu.VMEM((GW, TBLP), jnp.int32)] * 4
            + [pltpu.SemaphoreType.DMA] * 9),
    )
    def gather(table_hbm, idx_hbm, o_hbm, iall, b0, b1, b2, b3,
               sem_i, sg0, sg1, sg2, sg3, so0, so1, so2, so3):
        bufs = (b0, b1, b2, b3)
        sems_g = (sg0, sg1, sg2, sg3)
        sems_o = (so0, so1, so2, so3)
        core = lax.axis_index("core")
        sub = lax.axis_index("subcore")
        wid = sub * N_CORES + core
        lo = wid * nwin

        _start(idx_hbm.at[wid], iall, sem_i)
        _wait(idx_hbm.at[wid], iall, sem_i)

        def g_start(w, k):
            _start(table_hbm.at[iall.at[w]], bufs[k], sems_g[k])

        def g_wait(k):
            _wait(table_hbm.at[iall.at[0]], bufs[k], sems_g[k])

        def o_slice(w):
            return o_hbm.at[pl.ds((lo + w) * GW, GW), :]

        for k in range(4):
            g_start(k, k)

        nquads = nwin // 4

        @pl.loop(0, nquads)
        def _(q):
            w0 = 4 * q
            for k in range(4):
                g_wait(k)
                _start(bufs[k], o_slice(w0 + k), sems_o[k])
            for k in range(4):
                _wait(bufs[k], o_slice(w0 + k), sems_o[k])

                @pl.when(w0 + 4 + k < nwin)
                def _():
                    g_start(w0 + 4 + k, k)

        for w in range(4 * nquads, nwin):
            k = w % 4
            g_wait(k)
            pltpu.sync_copy(bufs[k], o_slice(w))

    return gather


_gather0 = _make_gather(CH0)
_gather1 = _make_gather(CH1)


def _mlp_body(g_ref, rt_ref, w1_ref, b1_ref, w2_ref, b2_ref, wd_ref, bd_ref,
              os_ref, oa_ref, ob_ref, oc_ref):
    u = g_ref[...]
    ga = lax.bitcast_convert_type(lax.shift_left(u, 16), jnp.float32)
    gb = lax.bitcast_convert_type(
        jnp.bitwise_and(u, jnp.int32(-65536)), jnp.float32)
    se = ga[:, :FEAT]
    h = se @ w1_ref[...] + b1_ref[0:1, :]
    h = h * (1.0 / (1.0 + jnp.exp(-h)))
    phi = h @ w2_ref[...] + b2_ref[0:1, :]

    rt = rt_ref[...].T
    x_ = rt[0:1, :]
    y_ = rt[1:2, :]
    z_ = rt[2:3, :]
    d2t = x_ * x_ + y_ * y_ + z_ * z_ + 3.0 * EPS
    dt = jnp.sqrt(d2t)
    inv_dt = 1.0 / dt
    th = (math.pi / CUTOFF) * dt
    s1 = jnp.sin(th) * inv_dt
    c2 = 2.0 * jnp.cos(th)
    rows = [s1]
    prev2 = jnp.zeros_like(s1)
    prev1 = s1
    for _ in range(N_RBF - 1):
        cur = c2 * prev1 - prev2
        rows.append(cur)
        prev2, prev1 = prev1, cur
    for _ in range(NB_PAD - N_RBF):
        rows.append(jnp.zeros_like(s1))
    rbf = jnp.concatenate(rows, axis=0).T
    ws = rbf @ wd_ref[...] + bd_ref[0:1, :]

    out = phi * ws
    s0 = out[:, 0:FEAT]
    s1o = out[:, FEAT:2 * FEAT]
    s2 = out[:, 2 * FEAT:3 * FEAT]

    os_ref[...] = s1o

    u8 = jnp.concatenate(
        [x_ * inv_dt, y_ * inv_dt, z_ * inv_dt] + [jnp.zeros_like(s1)] * 5,
        axis=0).T
    v_planes = (ga[:, FEAT:], gb[:, :FEAT], gb[:, FEAT:])
    for c, o_ref in enumerate((oa_ref, ob_ref, oc_ref)):
        o_ref[...] = s0 * v_planes[c] + s2 * u8[:, c:c + 1]


def _make_mlp(n_edges):
    plane = jax.ShapeDtypeStruct((n_edges, FEAT), jnp.float32)
    return pl.pallas_call(
        _mlp_body,
        grid=(n_edges // EDGE_BLK,),
        in_specs=[
            pl.BlockSpec((EDGE_BLK, TBLP), lambda i: (i, 0)),
            pl.BlockSpec((EDGE_BLK, 3), lambda i: (i, 0)),
            pl.BlockSpec((FEAT, FEAT), lambda i: (0, 0)),
            pl.BlockSpec((8, FEAT), lambda i: (0, 0)),
            pl.BlockSpec((FEAT, 3 * FEAT), lambda i: (0, 0)),
            pl.BlockSpec((8, 3 * FEAT), lambda i: (0, 0)),
            pl.BlockSpec((NB_PAD, 3 * FEAT), lambda i: (0, 0)),
            pl.BlockSpec((8, 3 * FEAT), lambda i: (0, 0)),
        ],
        out_specs=tuple(
            pl.BlockSpec((EDGE_BLK, FEAT), lambda i: (i, 0))
            for _ in range(4)),
        out_shape=(plane,) * 4,
    )


_mlp0 = _make_mlp(CH0)
_mlp1 = _make_mlp(CH1)


def _scatter_loop(in_slice, iall, acc, bufs, sems_in, sems_sc, nwin):
    def sc_start(w, k):
        _start(bufs[k], acc.at[iall.at[w]], sems_sc[k], add=True)

    def sc_wait(k):
        _wait(bufs[k], acc.at[iall.at[0]], sems_sc[k])

    def in_start(w, k):
        _start(in_slice(w), bufs[k], sems_in[k])

    def in_wait(w, k):
        _wait(in_slice(w), bufs[k], sems_in[k])

    nquads = nwin // 4

    @pl.loop(0, nquads)
    def _(q):
        w0 = 4 * q
        for k in range(4):
            in_wait(w0 + k, k)
            sc_start(w0 + k, k)
        for k in range(4):
            sc_wait(k)

            @pl.when(w0 + 4 + k < nwin)
            def _():
                in_start(w0 + 4 + k, k)

    for w in range(4 * nquads, nwin):
        k = w % 4
        in_wait(w, k)
        pltpu.sync_copy(bufs[k], acc.at[iall.at[w]], add=True)


_LROWS = N_NODES - (N_SUBCORES - 1) * (N_PAD // N_SUBCORES)


def _scatter_body(planes, inits, dst_hbm, outs, acc, iall, bufs,
                  sem_i, sems_in, sems_sc, nwin, final):
    core = lax.axis_index("core")
    sub = lax.axis_index("subcore")
    rows = N_PAD // N_SUBCORES
    rbase = sub * rows

    _start(dst_hbm.at[sub], iall, sem_i)

    def copy_out(o_hbm):
        if not final:
            pltpu.sync_copy(acc.at[pl.ds(rbase, rows)],
                            o_hbm.at[pl.ds(rbase, rows)])
            return

        @pl.when(sub < N_SUBCORES - 1)
        def _():
            pltpu.sync_copy(acc.at[pl.ds(rbase, rows)],
                            o_hbm.at[pl.ds(rbase, rows)])

        @pl.when(sub == N_SUBCORES - 1)
        def _():
            pltpu.sync_copy(acc.at[pl.ds(rbase, _LROWS)],
                            o_hbm.at[pl.ds(rbase, _LROWS)])

    def work(p_hbm, init_hbm, o_hbm, first):
        def in_slice(w):
            return p_hbm.at[pl.ds((sub * nwin + w) * SW, SW), :]

        for k in range(4):
            _start(in_slice(k), bufs[k], sems_in[k])
        pltpu.sync_copy(init_hbm.at[pl.ds(rbase, rows)],
                        acc.at[pl.ds(rbase, rows)])
        if first:
            _wait(dst_hbm.at[sub], iall, sem_i)
        plsc.subcore_barrier()
        _scatter_loop(in_slice, iall, acc, bufs, sems_in, sems_sc, nwin)
        plsc.subcore_barrier()
        copy_out(o_hbm)

    @pl.when(core == 0)
    def _():
        work(planes[0], inits[0], outs[0], True)

    @pl.when(core == 1)
    def _():
        work(planes[1], inits[1], outs[1], True)

    @pl.when(core == 0)
    def _():
        work(planes[2], inits[2], outs[2], False)

    @pl.when(core == 1)
    def _():
        work(planes[3], inits[3], outs[3], False)


def _make_scatter(n_edges, final):
    nwin = n_edges // (N_SUBCORES * SW)
    orows = N_NODES if final else N_PAD
    out_type = tuple(
        jax.ShapeDtypeStruct((orows, FEAT), jnp.float32) for _ in range(4))
    scratch = ([
        pltpu.VMEM_SHARED((N_PAD, FEAT), jnp.float32),
        pltpu.VMEM((nwin, SW), jnp.int32)]
        + [pltpu.VMEM((SW, FEAT), jnp.float32)] * 4
        + [pltpu.SemaphoreType.DMA] * 9)

    if final:
        @functools.partial(pl.kernel, out_type=out_type, mesh=_vector_mesh,
                           scratch_types=scratch)
        def sk(p0, p1, p2, p3, q0, q1, q2, q3, dst_hbm, o0, o1, o2, o3,
               acc, iall, b0, b1, b2, b3, sem_i,
               si0, si1, si2, si3, ss0, ss1, ss2, ss3):
            _scatter_body((p0, p1, p2, p3), (q0, q1, q2, q3), dst_hbm,
                          (o0, o1, o2, o3), acc, iall, (b0, b1, b2, b3),
                          sem_i, (si0, si1, si2, si3),
                          (ss0, ss1, ss2, ss3), nwin, True)
    else:
        @functools.partial(pl.kernel, out_type=out_type, mesh=_vector_mesh,
                           scratch_types=scratch)
        def sk(p0, p1, p2, p3, dst_hbm, z_hbm, o0, o1, o2, o3,
               acc, iall, b0, b1, b2, b3, sem_i,
               si0, si1, si2, si3, ss0, ss1, ss2, ss3):
            _scatter_body((p0, p1, p2, p3), (z_hbm,) * 4, dst_hbm,
                          (o0, o1, o2, o3), acc, iall, (b0, b1, b2, b3),
                          sem_i, (si0, si1, si2, si3),
                          (ss0, ss1, ss2, ss3), nwin, False)

    return sk


_scatter_first = _make_scatter(CH0, final=False)
_scatter_final = _make_scatter(CH1, final=True)


_PERM = np.concatenate([np.arange(FEAT) * 3,
                        np.arange(FEAT) * 3 + 1,
                        np.arange(FEAT) * 3 + 2])


def kernel(s_j, v_j, r_ij, nbrs, W1, b1, W2, b2, Wd, bd):
    tb16 = jnp.concatenate(
        [s_j, v_j[:, :, 0], v_j[:, :, 1], v_j[:, :, 2]],
        axis=1).astype(jnp.bfloat16)
    lo = lax.bitcast_convert_type(tb16[:, :TBLP], jnp.uint16)
    hi = lax.bitcast_convert_type(tb16[:, TBLP:], jnp.uint16)
    table = lax.bitcast_convert_type(
        jnp.bitwise_or(jnp.left_shift(hi.astype(jnp.uint32), 16),
                       lo.astype(jnp.uint32)), jnp.int32)
    src = nbrs[:, 1].astype(jnp.int32)
    dst = nbrs[:, 0].astype(jnp.int32)
    src0 = src[:CH0].reshape(N_WORKERS, CH0 // (N_WORKERS * GW), GW)
    src1 = src[CH0:].reshape(N_WORKERS, CH1 // (N_WORKERS * GW), GW)
    dst0 = dst[:CH0].reshape(N_SUBCORES, CH0 // (N_SUBCORES * SW), SW)
    dst1 = dst[CH0:].reshape(N_SUBCORES, CH1 // (N_SUBCORES * SW), SW)

    w2p = W2[:, _PERM]
    b2p = jnp.broadcast_to(b2[_PERM].reshape(1, -1), (8, 3 * FEAT))
    wdp = jnp.concatenate(
        [Wd[:, _PERM],
         jnp.zeros((NB_PAD - N_RBF, 3 * FEAT), jnp.float32)], axis=0)
    bdp = jnp.broadcast_to(bd[_PERM].reshape(1, -1), (8, 3 * FEAT))
    b1b = jnp.broadcast_to(b1.reshape(1, -1), (8, FEAT))

    zeros = jnp.zeros((N_PAD, FEAT), jnp.float32)

    g0 = _gather0(table, src0)
    g1 = _gather1(table, src1)
    m0 = _mlp0(g0, r_ij[:CH0], W1, b1b, w2p, b2p, wdp, bdp)
    m1 = _mlp1(g1, r_ij[CH0:], W1, b1b, w2p, b2p, wdp, bdp)
    parts = _scatter_first(m0[1], m0[2], m0[3], m0[0], dst0, zeros)
    dvx, dvy, dvz, ods = _scatter_final(
        m1[1], m1[2], m1[3], m1[0], *parts, dst1)

    return ods, jnp.stack([dvx, dvy, dvz], axis=-1)
